# Initial kernel scaffold; baseline (speedup 1.0000x reference)
import jax
import jax.numpy as jnp
from jax.experimental import pallas as pl


def kernel(a_idx, b_idx, values, coefficients):
    raise NotImplementedError("write your pallas kernel here")


# speedup vs baseline: 32.5228x; 32.5228x over previous
import functools

import jax
import jax.numpy as jnp
from jax import lax
from jax.experimental import pallas as pl
from jax.experimental.pallas import tpu as pltpu
from jax.experimental.pallas import tpu_sc as plsc

A = 4096
B = 4096
NC = 2
NS = 16
NW = NC * NS
LANES = 16


def _body(chunk, vsteps, a_hbm, b_hbm, v_hbm, tab_hbm, out_hbm,
          a_v, b_v, f_v, val_v, g_v, acc_v, sem):
    cid = lax.axis_index("c")
    sid = lax.axis_index("s")
    wid = sid * NC + cid
    base = wid * chunk

    cp_a = pltpu.async_copy(a_hbm.at[pl.ds(base, chunk)], a_v, sem)
    cp_b = pltpu.async_copy(b_hbm.at[pl.ds(base, chunk)], b_v, sem)
    cp_v = pltpu.async_copy(v_hbm.at[pl.ds(base, chunk)], val_v, sem)
    cp_a.wait()
    cp_b.wait()

    def idx_body(i, carry):
        s = pl.ds(i * LANES, LANES)
        f_v[s] = a_v[s] * B + b_v[s]
        return carry

    lax.fori_loop(0, vsteps, idx_body, 0)

    pltpu.async_copy(tab_hbm.at[f_v], g_v, sem).wait()
    cp_v.wait()

    def dot_body(i, acc):
        s = pl.ds(i * LANES, LANES)
        return acc + g_v[s] * val_v[s]

    acc = lax.fori_loop(0, vsteps, dot_body,
                        jnp.zeros((LANES,), jnp.float32))
    acc_v[...] = acc
    pltpu.sync_copy(acc_v, out_hbm.at[pl.ds(wid * LANES, LANES)])


def kernel(a_idx, b_idx, values, coefficients):
    n = a_idx.shape[0]
    vsteps = -(-n // (NW * LANES))
    chunk = vsteps * LANES
    padn = chunk * NW

    a_p = jnp.pad(a_idx, (0, padn - n))
    b_p = jnp.pad(b_idx, (0, padn - n))
    v_p = jnp.pad(values, (0, padn - n))
    tab = coefficients.reshape(A * B)

    mesh = plsc.VectorSubcoreMesh(core_axis_name="c", subcore_axis_name="s")
    out = pl.kernel(
        functools.partial(_body, chunk, vsteps),
        out_type=jax.ShapeDtypeStruct((NW * LANES,), jnp.float32),
        mesh=mesh,
        scratch_types=[
            pltpu.VMEM((chunk,), jnp.int32),
            pltpu.VMEM((chunk,), jnp.int32),
            pltpu.VMEM((chunk,), jnp.int32),
            pltpu.VMEM((chunk,), jnp.float32),
            pltpu.VMEM((chunk,), jnp.float32),
            pltpu.VMEM((LANES,), jnp.float32),
            pltpu.SemaphoreType.DMA,
        ],
    )(a_p, b_p, v_p, tab)
    return jnp.sum(out)
